# baseline (speedup 1.0000x reference)
import jax
import jax.numpy as jnp
from jax.experimental import pallas as pl


def kernel(x, We, be, Wd, bd):
    pre_act = jax.nn.relu(jax.lax.dot_general(
        x.astype(jnp.bfloat16), We.astype(jnp.bfloat16), (((1,), (1,)), ((), ())),
        preferred_element_type=jnp.float32) + be)
    topk_vals, topk_idx = jax.lax.top_k(pre_act, 64)
    rows = jnp.arange(pre_act.shape[0])[:, None]
    mask = jnp.zeros_like(pre_act).at[rows, topk_idx].set(1.0)
    z = pre_act * mask
    x_hat = z @ Wd.T + bd
    return (x_hat, z)


# speedup vs baseline: 11.3357x; 11.3357x over previous
import functools

import jax
import jax.numpy as jnp
from jax.experimental import pallas as pl

N_TOKENS = 2048
INPUT_DIM = 2048
HIDDEN_DIM = 16384
TOPK = 64

ENC_TH = 512
MSK_TM = 128
DEC_TH = 512


def _encode_kernel(x_ref, we_ref, be_ref, out_ref):
    xb = x_ref[...].astype(jnp.bfloat16)
    wb = we_ref[...].astype(jnp.bfloat16)
    acc = jax.lax.dot_general(xb, wb, (((1,), (1,)), ((), ())),
                              preferred_element_type=jnp.float32)
    out_ref[...] = jnp.maximum(acc + be_ref[...], 0.0)


def _topk_mask_kernel(p_ref, z_ref):
    v = p_ref[...]
    bits = jax.lax.bitcast_convert_type(v, jnp.int32)
    hi = jnp.max(bits, axis=1, keepdims=True)
    lo = jnp.zeros_like(hi)

    def body(_, carry):
        lo, hi = carry
        mid = lo + ((hi - lo + 1) >> 1)
        cnt = jnp.sum((bits >= mid).astype(jnp.int32), axis=1, keepdims=True)
        ge = cnt >= TOPK
        return jnp.where(ge, mid, lo), jnp.where(ge, hi, mid - 1)

    lo, hi = jax.lax.fori_loop(0, 31, body, (lo, hi))
    z_ref[...] = jnp.where(bits >= lo, v, 0.0)


def _decode_kernel(z_ref, wd_ref, bd_ref, out_ref):
    h = pl.program_id(0)
    zb = z_ref[...].astype(jnp.bfloat16)
    wb = wd_ref[...].astype(jnp.bfloat16)
    part = jax.lax.dot_general(zb, wb, (((1,), (1,)), ((), ())),
                               preferred_element_type=jnp.float32)

    @pl.when(h == 0)
    def _():
        out_ref[...] = part + bd_ref[...]

    @pl.when(h > 0)
    def _():
        out_ref[...] += part


@functools.partial(jax.jit, static_argnames=("interpret",))
def kernel(x, We, be, Wd, bd, interpret=False):
    pre_act = pl.pallas_call(
        _encode_kernel,
        grid=(HIDDEN_DIM // ENC_TH,),
        in_specs=[
            pl.BlockSpec((N_TOKENS, INPUT_DIM), lambda h: (0, 0)),
            pl.BlockSpec((ENC_TH, INPUT_DIM), lambda h: (h, 0)),
            pl.BlockSpec((ENC_TH,), lambda h: (h,)),
        ],
        out_specs=pl.BlockSpec((N_TOKENS, ENC_TH), lambda h: (0, h)),
        out_shape=jax.ShapeDtypeStruct((N_TOKENS, HIDDEN_DIM), jnp.float32),
        interpret=interpret,
    )(x, We, be)

    z = pl.pallas_call(
        _topk_mask_kernel,
        grid=(N_TOKENS // MSK_TM,),
        in_specs=[pl.BlockSpec((MSK_TM, HIDDEN_DIM), lambda r: (r, 0))],
        out_specs=pl.BlockSpec((MSK_TM, HIDDEN_DIM), lambda r: (r, 0)),
        out_shape=jax.ShapeDtypeStruct((N_TOKENS, HIDDEN_DIM), jnp.float32),
        interpret=interpret,
    )(pre_act)

    x_hat = pl.pallas_call(
        _decode_kernel,
        grid=(HIDDEN_DIM // DEC_TH,),
        in_specs=[
            pl.BlockSpec((N_TOKENS, DEC_TH), lambda h: (0, h)),
            pl.BlockSpec((INPUT_DIM, DEC_TH), lambda h: (0, h)),
            pl.BlockSpec((INPUT_DIM,), lambda h: (0,)),
        ],
        out_specs=pl.BlockSpec((N_TOKENS, INPUT_DIM), lambda h: (0, 0)),
        out_shape=jax.ShapeDtypeStruct((N_TOKENS, INPUT_DIM), jnp.float32),
        interpret=interpret,
    )(z, Wd, bd)

    return (x_hat, z)
